# baseline (speedup 1.0000x reference)
import functools

import jax
import jax.numpy as jnp
from jax import lax
from jax.experimental import pallas as pl
from jax.experimental.pallas import tpu as pltpu
from jax.experimental.pallas import tpu_sc as plsc

F32 = jnp.float32

E, NS, ND, D, H = 160000, 50000, 10000, 128, 128

NC, NSUB = 2, 16
NW = NC * NSUB

HP = 64
CB = 128
NBLK = E // CB
GJ = (NBLK + NW - 1) // NW
NBLK_CORE = NBLK // NC
SJ = (NBLK_CORE + NSUB - 1) // NSUB
RB = 80
NRB = ND // RB
RJ = (NRB + NSUB - 1) // NSUB


def _silu(x):
    return x * jax.nn.sigmoid(x)


def _ln(y, g, b):
    m = jnp.mean(y, axis=-1, keepdims=True)
    v = jnp.mean((y - m) ** 2, axis=-1, keepdims=True)
    return (y - m) * lax.rsqrt(v + 1e-5) * g + b


def _pack_bf16(y):
    u = lax.bitcast_convert_type(y, jnp.uint32)
    r = u + jnp.uint32(0x7FFF) + ((u >> jnp.uint32(16)) & jnp.uint32(1))
    lo = r[:, :64] >> jnp.uint32(16)
    hi = r[:, 64:] & jnp.uint32(0xFFFF0000)
    return lax.bitcast_convert_type(lo | hi, jnp.int32)


def _unpack_bf16(p):
    u = lax.bitcast_convert_type(p, jnp.uint32)
    lo = lax.bitcast_convert_type(u << jnp.uint32(16), F32)
    hi = lax.bitcast_convert_type(u & jnp.uint32(0xFFFF0000), F32)
    return jnp.concatenate([lo, hi], axis=-1)


def _grid_body(x_ref, w1b_ref, sw1_ref, sb1_ref, sw2_ref, sb2_ref, sg_ref,
               sbeta_ref, gp_ref, go_ref):
    x = x_ref[...]
    gp_ref[...] = _pack_bf16(jnp.dot(x, w1b_ref[...],
                                     preferred_element_type=F32))
    h = _silu(jnp.dot(x, sw1_ref[...], preferred_element_type=F32)
              + sb1_ref[...])
    y = jnp.dot(h, sw2_ref[...], preferred_element_type=F32) + sb2_ref[...]
    go_ref[...] = x + _ln(y, sg_ref[...], sbeta_ref[...])


def _meshproj_body(x_ref, w1c_ref, mp_ref):
    mp_ref[...] = _pack_bf16(jnp.dot(x_ref[...], w1c_ref[...],
                                     preferred_element_type=F32))


def _edge_body(g2m_ref, ga_ref, gb_ref, w1a_ref, b1_ref, w2_ref, b2_ref,
               g_ref, beta_ref, out_ref):
    x = (jnp.dot(g2m_ref[...], w1a_ref[...], preferred_element_type=F32)
         + _unpack_bf16(ga_ref[...]) + _unpack_bf16(gb_ref[...])
         + b1_ref[...])
    h = _silu(x)
    y = jnp.dot(h, w2_ref[...], preferred_element_type=F32) + b2_ref[...]
    out_ref[...] = _ln(y, g_ref[...], beta_ref[...])


def _dst_body(pr_ref, mesh_ref, dw1a_ref, dw1b_ref, db1_ref, dw2_ref,
              db2_ref, dg_ref, dbeta_ref, out_ref):
    p = pr_ref[...]
    agg = p[0] + p[1]
    mesh = mesh_ref[...]
    h = _silu(jnp.dot(agg, dw1a_ref[...], preferred_element_type=F32)
              + jnp.dot(mesh, dw1b_ref[...], preferred_element_type=F32)
              + db1_ref[...])
    y = jnp.dot(h, dw2_ref[...], preferred_element_type=F32) + db2_ref[...]
    out_ref[...] = mesh + _ln(y, dg_ref[...], dbeta_ref[...])


def _full(shape):
    nd = len(shape)
    return pl.BlockSpec(shape, lambda i: (0,) * nd)


def _sc_gather_body(sidx_hbm, didx_hbm, gp_hbm, mp_hbm, outg_hbm, outm_hbm,
                    sidx_v, didx_v, ga_v, gb_v, sem_a, sem_b):
    c = lax.axis_index("c")
    s = lax.axis_index("s")
    wid = s * NC + c

    def body(j, carry):
        blk = j * NW + wid

        @pl.when(blk < NBLK)
        def _():
            base = blk * CB
            pltpu.sync_copy(sidx_hbm.at[pl.ds(base, CB)], sidx_v)
            pltpu.sync_copy(didx_hbm.at[pl.ds(base, CB)], didx_v)
            cpa = pltpu.async_copy(gp_hbm.at[sidx_v], ga_v, sem_a)
            cpb = pltpu.async_copy(mp_hbm.at[didx_v], gb_v, sem_b)
            cpa.wait()
            cpb.wait()
            pltpu.sync_copy(ga_v, outg_hbm.at[pl.ds(base, CB)])
            pltpu.sync_copy(gb_v, outm_hbm.at[pl.ds(base, CB)])

        return carry

    lax.fori_loop(0, GJ, body, 0)


def _sc_scatter_body(didx_hbm, ef_hbm, out_hbm, idx_v, rows_v, zbuf_v,
                     acc_sh):
    c = lax.axis_index("c")
    s = lax.axis_index("s")

    zero = jnp.zeros((16,), F32)

    def zrow(r, carry):
        for k in range(8):
            zbuf_v[r, pl.ds(k * 16, 16)] = zero
        return carry

    lax.fori_loop(0, RB, zrow, 0)

    def zcopy(q, carry):
        rblk = q * NSUB + s

        @pl.when(rblk < NRB)
        def _():
            pltpu.sync_copy(zbuf_v, acc_sh.at[pl.ds(rblk * RB, RB)])

        return carry

    lax.fori_loop(0, RJ, zcopy, 0)
    plsc.subcore_barrier()

    def body(j, carry):
        t = j * NSUB + s

        @pl.when(t < NBLK_CORE)
        def _():
            blk = c * NBLK_CORE + t
            base = blk * CB
            pltpu.sync_copy(didx_hbm.at[pl.ds(base, CB)], idx_v)
            pltpu.sync_copy(ef_hbm.at[pl.ds(base, CB)], rows_v)
            pltpu.sync_copy(rows_v, acc_sh.at[idx_v], add=True)

        return carry

    lax.fori_loop(0, SJ, body, 0)
    plsc.subcore_barrier()

    def drain(q, carry):
        rblk = q * NSUB + s

        @pl.when(rblk < NRB)
        def _():
            pltpu.sync_copy(acc_sh.at[pl.ds(rblk * RB, RB)],
                            out_hbm.at[c].at[pl.ds(rblk * RB, RB)])

        return carry

    lax.fori_loop(0, RJ, drain, 0)


def kernel(g2m_efeat, grid_nfeat, mesh_nfeat, src_idx, dst_idx,
           edge_W1, edge_b1, edge_W2, edge_b2, edge_g, edge_beta,
           dst_W1, dst_b1, dst_W2, dst_b2, dst_g, dst_beta,
           src_W1, src_b1, src_W2, src_b2, src_g, src_beta):
    W1a, W1b, W1c = edge_W1[:D], edge_W1[D:2 * D], edge_W1[2 * D:]
    dW1a, dW1b = dst_W1[:D], dst_W1[D:]
    r2 = lambda v: v.reshape(1, D)

    GB = 2000
    grid_proj, grid_out = pl.pallas_call(
        _grid_body,
        grid=(NS // GB,),
        in_specs=[pl.BlockSpec((GB, D), lambda i: (i, 0)),
                  _full((D, H)), _full((D, H)), _full((1, H)),
                  _full((H, D)), _full((1, D)), _full((1, D)),
                  _full((1, D))],
        out_specs=[pl.BlockSpec((GB, HP), lambda i: (i, 0)),
                   pl.BlockSpec((GB, D), lambda i: (i, 0))],
        out_shape=[jax.ShapeDtypeStruct((NS, HP), jnp.int32),
                   jax.ShapeDtypeStruct((NS, D), F32)],
    )(grid_nfeat, W1b, src_W1, r2(src_b1), src_W2, r2(src_b2),
      r2(src_g), r2(src_beta))

    MB = 1000
    mesh_proj = pl.pallas_call(
        _meshproj_body,
        grid=(ND // MB,),
        in_specs=[pl.BlockSpec((MB, D), lambda i: (i, 0)), _full((D, H))],
        out_specs=pl.BlockSpec((MB, HP), lambda i: (i, 0)),
        out_shape=jax.ShapeDtypeStruct((ND, HP), jnp.int32),
    )(mesh_nfeat, W1c)

    mesh = plsc.VectorSubcoreMesh(core_axis_name="c", subcore_axis_name="s",
                                  num_cores=NC, num_subcores=NSUB)
    ga, gb = pl.kernel(
        _sc_gather_body,
        out_type=[jax.ShapeDtypeStruct((E, HP), jnp.int32),
                  jax.ShapeDtypeStruct((E, HP), jnp.int32)],
        mesh=mesh,
        compiler_params=pltpu.CompilerParams(use_tc_tiling_on_sc=False),
        scratch_types=[pltpu.VMEM((CB,), jnp.int32),
                       pltpu.VMEM((CB,), jnp.int32),
                       pltpu.VMEM((CB, HP), jnp.int32),
                       pltpu.VMEM((CB, HP), jnp.int32),
                       pltpu.SemaphoreType.DMA,
                       pltpu.SemaphoreType.DMA],
    )(src_idx, dst_idx, grid_proj, mesh_proj)

    EB = 1600
    efeat = pl.pallas_call(
        _edge_body,
        grid=(E // EB,),
        in_specs=[pl.BlockSpec((EB, D), lambda i: (i, 0)),
                  pl.BlockSpec((EB, HP), lambda i: (i, 0)),
                  pl.BlockSpec((EB, HP), lambda i: (i, 0)),
                  _full((D, H)), _full((1, H)), _full((H, D)),
                  _full((1, D)), _full((1, D)), _full((1, D))],
        out_specs=pl.BlockSpec((EB, D), lambda i: (i, 0)),
        out_shape=jax.ShapeDtypeStruct((E, D), F32),
    )(g2m_efeat, ga, gb, W1a, r2(edge_b1), edge_W2, r2(edge_b2),
      r2(edge_g), r2(edge_beta))

    partials = pl.kernel(
        _sc_scatter_body,
        out_type=jax.ShapeDtypeStruct((NC, ND, D), F32),
        mesh=mesh,
        scratch_types=[pltpu.VMEM((CB,), jnp.int32),
                       pltpu.VMEM((CB, D), F32),
                       pltpu.VMEM((RB, D), F32),
                       pltpu.VMEM_SHARED((ND, D), F32)],
    )(dst_idx, efeat)

    DBK = 1000
    mesh_out = pl.pallas_call(
        _dst_body,
        grid=(ND // DBK,),
        in_specs=[pl.BlockSpec((NC, DBK, D), lambda i: (0, i, 0)),
                  pl.BlockSpec((DBK, D), lambda i: (i, 0)),
                  _full((D, H)), _full((D, H)), _full((1, H)),
                  _full((H, D)), _full((1, D)), _full((1, D)),
                  _full((1, D))],
        out_specs=pl.BlockSpec((DBK, D), lambda i: (i, 0)),
        out_shape=jax.ShapeDtypeStruct((ND, D), F32),
    )(partials, mesh_nfeat, dW1a, dW1b, r2(dst_b1), dst_W2, r2(dst_b2),
      r2(dst_g), r2(dst_beta))

    return (grid_out, mesh_out)


# speedup vs baseline: 1.5098x; 1.5098x over previous
import functools

import jax
import jax.numpy as jnp
from jax import lax
from jax.experimental import pallas as pl
from jax.experimental.pallas import tpu as pltpu
from jax.experimental.pallas import tpu_sc as plsc

F32 = jnp.float32

E, NS, ND, D, H = 160000, 50000, 10000, 128, 128

NC, NSUB = 2, 16
NW = NC * NSUB

CB = 128
NBLK = E // CB
GJ2 = (NBLK + 2 * NW - 1) // (2 * NW)
NBLK_CORE = NBLK // NC
SJ2 = (NBLK_CORE + 2 * NSUB - 1) // (2 * NSUB)
RB = 80
NRB = ND // RB
RJ = (NRB + NSUB - 1) // NSUB


def _silu(x):
    return x * jax.nn.sigmoid(x)


def _ln(y, g, b):
    m = jnp.mean(y, axis=-1, keepdims=True)
    v = jnp.mean((y - m) ** 2, axis=-1, keepdims=True)
    return (y - m) * lax.rsqrt(v + 1e-5) * g + b


def _grid_body(x_ref, w1b_ref, sw1_ref, sb1_ref, sw2_ref, sb2_ref, sg_ref,
               sbeta_ref, gp_ref, go_ref):
    x = x_ref[...]
    gp_ref[...] = jnp.dot(x, w1b_ref[...], preferred_element_type=F32)
    h = _silu(jnp.dot(x, sw1_ref[...], preferred_element_type=F32)
              + sb1_ref[...])
    y = jnp.dot(h, sw2_ref[...], preferred_element_type=F32) + sb2_ref[...]
    go_ref[...] = x + _ln(y, sg_ref[...], sbeta_ref[...])


def _meshproj_body(x_ref, w1c_ref, mp_ref):
    mp_ref[...] = jnp.dot(x_ref[...], w1c_ref[...],
                          preferred_element_type=F32)


def _edge_body(g2m_ref, ga_ref, gb_ref, w1a_ref, b1_ref, w2_ref, b2_ref,
               g_ref, beta_ref, out_ref):
    x = (jnp.dot(g2m_ref[...], w1a_ref[...], preferred_element_type=F32)
         + ga_ref[...] + gb_ref[...] + b1_ref[...])
    h = _silu(x)
    y = jnp.dot(h, w2_ref[...], preferred_element_type=F32) + b2_ref[...]
    out_ref[...] = _ln(y, g_ref[...], beta_ref[...])


def _dst_body(pr_ref, mesh_ref, dw1a_ref, dw1b_ref, db1_ref, dw2_ref,
              db2_ref, dg_ref, dbeta_ref, out_ref):
    p = pr_ref[...]
    agg = p[0] + p[1]
    mesh = mesh_ref[...]
    h = _silu(jnp.dot(agg, dw1a_ref[...], preferred_element_type=F32)
              + jnp.dot(mesh, dw1b_ref[...], preferred_element_type=F32)
              + db1_ref[...])
    y = jnp.dot(h, dw2_ref[...], preferred_element_type=F32) + db2_ref[...]
    out_ref[...] = mesh + _ln(y, dg_ref[...], dbeta_ref[...])


def _full(shape):
    nd = len(shape)
    return pl.BlockSpec(shape, lambda i: (0,) * nd)


def _sc_gather_body(sidx_hbm, didx_hbm, gp_hbm, mp_hbm, outg_hbm, outm_hbm,
                    si0, di0, ga0, gb0, si1, di1, ga1, gb1,
                    smi0, smg0, smo0, smi1, smg1, smo1):
    c = lax.axis_index("c")
    s = lax.axis_index("s")
    wid = s * NC + c
    slots = ((si0, di0, ga0, gb0, smi0, smg0, smo0),
             (si1, di1, ga1, gb1, smi1, smg1, smo1))

    def blk_of(q, sl):
        return (q * 2 + sl) * NW + wid

    def fire_idx(q, sl):
        siv, div, _, _, smi, _, _ = slots[sl]
        blk = blk_of(q, sl)

        @pl.when(blk < NBLK)
        def _():
            base = blk * CB
            pltpu.async_copy(sidx_hbm.at[pl.ds(base, CB)], siv, smi)
            pltpu.async_copy(didx_hbm.at[pl.ds(base, CB)], div, smi)

    fire_idx(0, 0)
    fire_idx(0, 1)

    def body(q, carry):
        for sl in (0, 1):
            siv, div, gav, gbv, smi, smg, smo = slots[sl]
            blk = blk_of(q, sl)
            prev = blk - 2 * NW

            @pl.when(jnp.logical_and(prev >= 0, prev < NBLK))
            def _():
                pltpu.make_async_copy(gav, outg_hbm.at[pl.ds(0, CB)],
                                      smo).wait()
                pltpu.make_async_copy(gbv, outm_hbm.at[pl.ds(0, CB)],
                                      smo).wait()

            @pl.when(blk < NBLK)
            def _():
                pltpu.make_async_copy(sidx_hbm.at[pl.ds(0, CB)], siv,
                                      smi).wait()
                pltpu.make_async_copy(didx_hbm.at[pl.ds(0, CB)], div,
                                      smi).wait()
                pltpu.async_copy(gp_hbm.at[siv], gav, smg)
                pltpu.async_copy(mp_hbm.at[div], gbv, smg)

        for sl in (0, 1):
            siv, div, gav, gbv, smi, smg, smo = slots[sl]
            blk = blk_of(q, sl)

            @pl.when(blk < NBLK)
            def _():
                base = blk * CB
                pltpu.make_async_copy(gp_hbm.at[siv], gav, smg).wait()
                pltpu.make_async_copy(mp_hbm.at[div], gbv, smg).wait()
                pltpu.async_copy(gav, outg_hbm.at[pl.ds(base, CB)], smo)
                pltpu.async_copy(gbv, outm_hbm.at[pl.ds(base, CB)], smo)

            fire_idx(q + 1, sl)
        return carry

    lax.fori_loop(0, GJ2, body, 0)

    for sl in (0, 1):
        siv, div, gav, gbv, smi, smg, smo = slots[sl]
        blk = blk_of(GJ2 - 1, sl)

        @pl.when(blk < NBLK)
        def _():
            pltpu.make_async_copy(gav, outg_hbm.at[pl.ds(0, CB)], smo).wait()
            pltpu.make_async_copy(gbv, outm_hbm.at[pl.ds(0, CB)], smo).wait()


def _sc_scatter_body(didx_hbm, ef_hbm, out_hbm, idx0, rows0, idx1, rows1,
                     zbuf_v, acc_sh, smf0, smf1):
    c = lax.axis_index("c")
    s = lax.axis_index("s")
    slots = ((idx0, rows0, smf0), (idx1, rows1, smf1))

    zero = jnp.zeros((16,), F32)

    def zrow(r, carry):
        for k in range(8):
            zbuf_v[r, pl.ds(k * 16, 16)] = zero
        return carry

    lax.fori_loop(0, RB, zrow, 0)

    def zcopy(q, carry):
        rblk = q * NSUB + s

        @pl.when(rblk < NRB)
        def _():
            pltpu.sync_copy(zbuf_v, acc_sh.at[pl.ds(rblk * RB, RB)])

        return carry

    lax.fori_loop(0, RJ, zcopy, 0)
    plsc.subcore_barrier()

    def blk_of(q, sl):
        return (q * 2 + sl) * NSUB + s

    def fire_fetch(q, sl):
        idxv, rowsv, smf = slots[sl]
        t = blk_of(q, sl)

        @pl.when(t < NBLK_CORE)
        def _():
            base = (c * NBLK_CORE + t) * CB
            pltpu.async_copy(didx_hbm.at[pl.ds(base, CB)], idxv, smf)
            pltpu.async_copy(ef_hbm.at[pl.ds(base, CB)], rowsv, smf)

    fire_fetch(0, 0)
    fire_fetch(0, 1)

    def body(q, carry):
        for sl in (0, 1):
            idxv, rowsv, smf = slots[sl]
            t = blk_of(q, sl)

            @pl.when(t < NBLK_CORE)
            def _():
                pltpu.make_async_copy(didx_hbm.at[pl.ds(0, CB)], idxv,
                                      smf).wait()
                pltpu.make_async_copy(ef_hbm.at[pl.ds(0, CB)], rowsv,
                                      smf).wait()
                pltpu.sync_copy(rowsv, acc_sh.at[idxv], add=True)

            fire_fetch(q + 1, sl)
        return carry

    lax.fori_loop(0, SJ2, body, 0)
    plsc.subcore_barrier()

    def drain(q, carry):
        rblk = q * NSUB + s

        @pl.when(rblk < NRB)
        def _():
            pltpu.sync_copy(acc_sh.at[pl.ds(rblk * RB, RB)],
                            out_hbm.at[c].at[pl.ds(rblk * RB, RB)])

        return carry

    lax.fori_loop(0, RJ, drain, 0)


def kernel(g2m_efeat, grid_nfeat, mesh_nfeat, src_idx, dst_idx,
           edge_W1, edge_b1, edge_W2, edge_b2, edge_g, edge_beta,
           dst_W1, dst_b1, dst_W2, dst_b2, dst_g, dst_beta,
           src_W1, src_b1, src_W2, src_b2, src_g, src_beta):
    W1a, W1b, W1c = edge_W1[:D], edge_W1[D:2 * D], edge_W1[2 * D:]
    dW1a, dW1b = dst_W1[:D], dst_W1[D:]
    r2 = lambda v: v.reshape(1, D)

    GB = 2000
    grid_proj, grid_out = pl.pallas_call(
        _grid_body,
        grid=(NS // GB,),
        in_specs=[pl.BlockSpec((GB, D), lambda i: (i, 0)),
                  _full((D, H)), _full((D, H)), _full((1, H)),
                  _full((H, D)), _full((1, D)), _full((1, D)),
                  _full((1, D))],
        out_specs=[pl.BlockSpec((GB, H), lambda i: (i, 0)),
                   pl.BlockSpec((GB, D), lambda i: (i, 0))],
        out_shape=[jax.ShapeDtypeStruct((NS, H), F32),
                   jax.ShapeDtypeStruct((NS, D), F32)],
    )(grid_nfeat, W1b, src_W1, r2(src_b1), src_W2, r2(src_b2),
      r2(src_g), r2(src_beta))

    MB = 1000
    mesh_proj = pl.pallas_call(
        _meshproj_body,
        grid=(ND // MB,),
        in_specs=[pl.BlockSpec((MB, D), lambda i: (i, 0)), _full((D, H))],
        out_specs=pl.BlockSpec((MB, H), lambda i: (i, 0)),
        out_shape=jax.ShapeDtypeStruct((ND, H), F32),
    )(mesh_nfeat, W1c)

    mesh = plsc.VectorSubcoreMesh(core_axis_name="c", subcore_axis_name="s",
                                  num_cores=NC, num_subcores=NSUB)
    ga, gb = pl.kernel(
        _sc_gather_body,
        out_type=[jax.ShapeDtypeStruct((E, H), F32),
                  jax.ShapeDtypeStruct((E, H), F32)],
        mesh=mesh,
        scratch_types=[pltpu.VMEM((CB,), jnp.int32),
                       pltpu.VMEM((CB,), jnp.int32),
                       pltpu.VMEM((CB, H), F32),
                       pltpu.VMEM((CB, H), F32),
                       pltpu.VMEM((CB,), jnp.int32),
                       pltpu.VMEM((CB,), jnp.int32),
                       pltpu.VMEM((CB, H), F32),
                       pltpu.VMEM((CB, H), F32),
                       pltpu.SemaphoreType.DMA,
                       pltpu.SemaphoreType.DMA,
                       pltpu.SemaphoreType.DMA,
                       pltpu.SemaphoreType.DMA,
                       pltpu.SemaphoreType.DMA,
                       pltpu.SemaphoreType.DMA],
    )(src_idx, dst_idx, grid_proj, mesh_proj)

    EB = 1600
    efeat = pl.pallas_call(
        _edge_body,
        grid=(E // EB,),
        in_specs=[pl.BlockSpec((EB, D), lambda i: (i, 0)),
                  pl.BlockSpec((EB, H), lambda i: (i, 0)),
                  pl.BlockSpec((EB, H), lambda i: (i, 0)),
                  _full((D, H)), _full((1, H)), _full((H, D)),
                  _full((1, D)), _full((1, D)), _full((1, D))],
        out_specs=pl.BlockSpec((EB, D), lambda i: (i, 0)),
        out_shape=jax.ShapeDtypeStruct((E, D), F32),
    )(g2m_efeat, ga, gb, W1a, r2(edge_b1), edge_W2, r2(edge_b2),
      r2(edge_g), r2(edge_beta))

    partials = pl.kernel(
        _sc_scatter_body,
        out_type=jax.ShapeDtypeStruct((NC, ND, D), F32),
        mesh=mesh,
        scratch_types=[pltpu.VMEM((CB,), jnp.int32),
                       pltpu.VMEM((CB, D), F32),
                       pltpu.VMEM((CB,), jnp.int32),
                       pltpu.VMEM((CB, D), F32),
                       pltpu.VMEM((RB, D), F32),
                       pltpu.VMEM_SHARED((ND, D), F32),
                       pltpu.SemaphoreType.DMA,
                       pltpu.SemaphoreType.DMA],
    )(dst_idx, efeat)

    DBK = 1000
    mesh_out = pl.pallas_call(
        _dst_body,
        grid=(ND // DBK,),
        in_specs=[pl.BlockSpec((NC, DBK, D), lambda i: (0, i, 0)),
                  pl.BlockSpec((DBK, D), lambda i: (i, 0)),
                  _full((D, H)), _full((D, H)), _full((1, H)),
                  _full((H, D)), _full((1, D)), _full((1, D)),
                  _full((1, D))],
        out_specs=pl.BlockSpec((DBK, D), lambda i: (i, 0)),
        out_shape=jax.ShapeDtypeStruct((ND, D), F32),
    )(partials, mesh_nfeat, dW1a, dW1b, r2(dst_b1), dst_W2, r2(dst_b2),
      r2(dst_g), r2(dst_beta))

    return (grid_out, mesh_out)


# speedup vs baseline: 1.5641x; 1.0360x over previous
import functools

import jax
import jax.numpy as jnp
from jax import lax
from jax.experimental import pallas as pl
from jax.experimental.pallas import tpu as pltpu
from jax.experimental.pallas import tpu_sc as plsc

F32 = jnp.float32

E, NS, ND, D, H = 160000, 50000, 10000, 128, 128

NC, NSUB = 2, 16
NW = NC * NSUB

CB = 128
NBLK = E // CB
SPLIT = 640
RB = 80
NRB = ND // RB
RJ = (NRB + NSUB - 1) // NSUB


def _silu(x):
    return x * jax.nn.sigmoid(x)


def _ln(y, g, b):
    m = jnp.mean(y, axis=-1, keepdims=True)
    v = jnp.mean((y - m) ** 2, axis=-1, keepdims=True)
    return (y - m) * lax.rsqrt(v + 1e-5) * g + b


def _grid_body(x_ref, w1b_ref, sw1_ref, sb1_ref, sw2_ref, sb2_ref, sg_ref,
               sbeta_ref, gp_ref, go_ref):
    x = x_ref[...]
    gp_ref[...] = jnp.dot(x, w1b_ref[...], preferred_element_type=F32)
    h = _silu(jnp.dot(x, sw1_ref[...], preferred_element_type=F32)
              + sb1_ref[...])
    y = jnp.dot(h, sw2_ref[...], preferred_element_type=F32) + sb2_ref[...]
    go_ref[...] = x + _ln(y, sg_ref[...], sbeta_ref[...])


def _meshproj_body(x_ref, w1c_ref, mp_ref):
    mp_ref[...] = jnp.dot(x_ref[...], w1c_ref[...],
                          preferred_element_type=F32)


def _edge_body(g2m_ref, ga_ref, gb_ref, w1a_ref, b1_ref, w2_ref, b2_ref,
               g_ref, beta_ref, out_ref):
    x = (jnp.dot(g2m_ref[...], w1a_ref[...], preferred_element_type=F32)
         + ga_ref[...] + gb_ref[...] + b1_ref[...])
    h = _silu(x)
    y = jnp.dot(h, w2_ref[...], preferred_element_type=F32) + b2_ref[...]
    out_ref[...] = _ln(y, g_ref[...], beta_ref[...])


def _dst_body(p0_ref, p1_ref, mesh_ref, dw1a_ref, dw1b_ref, db1_ref,
              dw2_ref, db2_ref, dg_ref, dbeta_ref, out_ref):
    p0 = p0_ref[...]
    p1 = p1_ref[...]
    agg = p0[0] + p0[1] + p1[0] + p1[1]
    mesh = mesh_ref[...]
    h = _silu(jnp.dot(agg, dw1a_ref[...], preferred_element_type=F32)
              + jnp.dot(mesh, dw1b_ref[...], preferred_element_type=F32)
              + db1_ref[...])
    y = jnp.dot(h, dw2_ref[...], preferred_element_type=F32) + db2_ref[...]
    out_ref[...] = mesh + _ln(y, dg_ref[...], dbeta_ref[...])


def _full(shape):
    nd = len(shape)
    return pl.BlockSpec(shape, lambda i: (0,) * nd)


def _sc_gather_body(blo, bhi, sidx_hbm, didx_hbm, gp_hbm, mp_hbm,
                    outg_hbm, outm_hbm,
                    si0, di0, ga0, gb0, si1, di1, ga1, gb1,
                    smi0, smg0, smo0, smi1, smg1, smo1):
    c = lax.axis_index("c")
    s = lax.axis_index("s")
    wid = s * NC + c
    slots = ((si0, di0, ga0, gb0, smi0, smg0, smo0),
             (si1, di1, ga1, gb1, smi1, smg1, smo1))
    nblk = bhi - blo
    gj2 = (nblk + 2 * NW - 1) // (2 * NW)

    def blk_of(q, sl):
        return (q * 2 + sl) * NW + wid

    def fire_idx(q, sl):
        siv, div, _, _, smi, _, _ = slots[sl]
        blk = blk_of(q, sl)

        @pl.when(blk < nblk)
        def _():
            base = (blo + blk) * CB
            pltpu.async_copy(sidx_hbm.at[pl.ds(base, CB)], siv, smi)
            pltpu.async_copy(didx_hbm.at[pl.ds(base, CB)], div, smi)

    fire_idx(0, 0)
    fire_idx(0, 1)

    def body(q, carry):
        for sl in (0, 1):
            siv, div, gav, gbv, smi, smg, smo = slots[sl]
            blk = blk_of(q, sl)
            prev = blk - 2 * NW

            @pl.when(jnp.logical_and(prev >= 0, prev < nblk))
            def _():
                pltpu.make_async_copy(gav, outg_hbm.at[pl.ds(0, CB)],
                                      smo).wait()
                pltpu.make_async_copy(gbv, outm_hbm.at[pl.ds(0, CB)],
                                      smo).wait()

            @pl.when(blk < nblk)
            def _():
                pltpu.make_async_copy(sidx_hbm.at[pl.ds(0, CB)], siv,
                                      smi).wait()
                pltpu.make_async_copy(didx_hbm.at[pl.ds(0, CB)], div,
                                      smi).wait()
                pltpu.async_copy(gp_hbm.at[siv], gav, smg)
                pltpu.async_copy(mp_hbm.at[div], gbv, smg)

        for sl in (0, 1):
            siv, div, gav, gbv, smi, smg, smo = slots[sl]
            blk = blk_of(q, sl)

            @pl.when(blk < nblk)
            def _():
                base = blk * CB
                pltpu.make_async_copy(gp_hbm.at[siv], gav, smg).wait()
                pltpu.make_async_copy(mp_hbm.at[div], gbv, smg).wait()
                pltpu.async_copy(gav, outg_hbm.at[pl.ds(base, CB)], smo)
                pltpu.async_copy(gbv, outm_hbm.at[pl.ds(base, CB)], smo)

            fire_idx(q + 1, sl)
        return carry

    lax.fori_loop(0, gj2, body, 0)

    for sl in (0, 1):
        siv, div, gav, gbv, smi, smg, smo = slots[sl]
        blk = blk_of(gj2 - 1, sl)

        @pl.when(blk < nblk)
        def _():
            pltpu.make_async_copy(gav, outg_hbm.at[pl.ds(0, CB)], smo).wait()
            pltpu.make_async_copy(gbv, outm_hbm.at[pl.ds(0, CB)], smo).wait()


def _sc_scatter_body(blo, bhi, didx_hbm, ef_hbm, out_hbm,
                     idx0, rows0, idx1, rows1, zbuf_v, acc_sh, smf0, smf1):
    c = lax.axis_index("c")
    s = lax.axis_index("s")
    slots = ((idx0, rows0, smf0), (idx1, rows1, smf1))
    percore = (bhi - blo) // NC
    sj2 = (percore + 2 * NSUB - 1) // (2 * NSUB)

    zero = jnp.zeros((16,), F32)

    def zrow(r, carry):
        for k in range(8):
            zbuf_v[r, pl.ds(k * 16, 16)] = zero
        return carry

    lax.fori_loop(0, RB, zrow, 0)

    def zcopy(q, carry):
        rblk = q * NSUB + s

        @pl.when(rblk < NRB)
        def _():
            pltpu.sync_copy(zbuf_v, acc_sh.at[pl.ds(rblk * RB, RB)])

        return carry

    lax.fori_loop(0, RJ, zcopy, 0)
    plsc.subcore_barrier()

    def blk_of(q, sl):
        return (q * 2 + sl) * NSUB + s

    def fire_fetch(q, sl):
        idxv, rowsv, smf = slots[sl]
        t = blk_of(q, sl)

        @pl.when(t < percore)
        def _():
            local = c * percore + t
            pltpu.async_copy(
                didx_hbm.at[pl.ds((blo + local) * CB, CB)], idxv, smf)
            pltpu.async_copy(ef_hbm.at[pl.ds(local * CB, CB)], rowsv, smf)

    fire_fetch(0, 0)
    fire_fetch(0, 1)

    def body(q, carry):
        for sl in (0, 1):
            idxv, rowsv, smf = slots[sl]
            t = blk_of(q, sl)

            @pl.when(t < percore)
            def _():
                pltpu.make_async_copy(didx_hbm.at[pl.ds(0, CB)], idxv,
                                      smf).wait()
                pltpu.make_async_copy(ef_hbm.at[pl.ds(0, CB)], rowsv,
                                      smf).wait()
                pltpu.sync_copy(rowsv, acc_sh.at[idxv], add=True)

            fire_fetch(q + 1, sl)
        return carry

    lax.fori_loop(0, sj2, body, 0)
    plsc.subcore_barrier()

    def drain(q, carry):
        rblk = q * NSUB + s

        @pl.when(rblk < NRB)
        def _():
            pltpu.sync_copy(acc_sh.at[pl.ds(rblk * RB, RB)],
                            out_hbm.at[c].at[pl.ds(rblk * RB, RB)])

        return carry

    lax.fori_loop(0, RJ, drain, 0)


def kernel(g2m_efeat, grid_nfeat, mesh_nfeat, src_idx, dst_idx,
           edge_W1, edge_b1, edge_W2, edge_b2, edge_g, edge_beta,
           dst_W1, dst_b1, dst_W2, dst_b2, dst_g, dst_beta,
           src_W1, src_b1, src_W2, src_b2, src_g, src_beta):
    W1a, W1b, W1c = edge_W1[:D], edge_W1[D:2 * D], edge_W1[2 * D:]
    dW1a, dW1b = dst_W1[:D], dst_W1[D:]
    r2 = lambda v: v.reshape(1, D)

    GB = 2000
    grid_proj, grid_out = pl.pallas_call(
        _grid_body,
        grid=(NS // GB,),
        in_specs=[pl.BlockSpec((GB, D), lambda i: (i, 0)),
                  _full((D, H)), _full((D, H)), _full((1, H)),
                  _full((H, D)), _full((1, D)), _full((1, D)),
                  _full((1, D))],
        out_specs=[pl.BlockSpec((GB, H), lambda i: (i, 0)),
                   pl.BlockSpec((GB, D), lambda i: (i, 0))],
        out_shape=[jax.ShapeDtypeStruct((NS, H), F32),
                   jax.ShapeDtypeStruct((NS, D), F32)],
    )(grid_nfeat, W1b, src_W1, r2(src_b1), src_W2, r2(src_b2),
      r2(src_g), r2(src_beta))

    MB = 1000
    mesh_proj = pl.pallas_call(
        _meshproj_body,
        grid=(ND // MB,),
        in_specs=[pl.BlockSpec((MB, D), lambda i: (i, 0)), _full((D, H))],
        out_specs=pl.BlockSpec((MB, H), lambda i: (i, 0)),
        out_shape=jax.ShapeDtypeStruct((ND, H), F32),
    )(mesh_nfeat, W1c)

    mesh = plsc.VectorSubcoreMesh(core_axis_name="c", subcore_axis_name="s",
                                  num_cores=NC, num_subcores=NSUB)

    def gather_half(blo, bhi):
        n = (bhi - blo) * CB
        return pl.kernel(
            functools.partial(_sc_gather_body, blo, bhi),
            out_type=[jax.ShapeDtypeStruct((n, H), F32),
                      jax.ShapeDtypeStruct((n, H), F32)],
            mesh=mesh,
            scratch_types=[pltpu.VMEM((CB,), jnp.int32),
                           pltpu.VMEM((CB,), jnp.int32),
                           pltpu.VMEM((CB, H), F32),
                           pltpu.VMEM((CB, H), F32),
                           pltpu.VMEM((CB,), jnp.int32),
                           pltpu.VMEM((CB,), jnp.int32),
                           pltpu.VMEM((CB, H), F32),
                           pltpu.VMEM((CB, H), F32)]
                          + [pltpu.SemaphoreType.DMA] * 6,
        )(src_idx, dst_idx, grid_proj, mesh_proj)

    EB = 1280

    def edge_half(blo, bhi, ga, gb):
        n = (bhi - blo) * CB
        goff = blo * CB // EB
        return pl.pallas_call(
            _edge_body,
            grid=(n // EB,),
            in_specs=[pl.BlockSpec((EB, D), lambda i: (i + goff, 0)),
                      pl.BlockSpec((EB, H), lambda i: (i, 0)),
                      pl.BlockSpec((EB, H), lambda i: (i, 0)),
                      _full((D, H)), _full((1, H)), _full((H, D)),
                      _full((1, D)), _full((1, D)), _full((1, D))],
            out_specs=pl.BlockSpec((EB, D), lambda i: (i, 0)),
            out_shape=jax.ShapeDtypeStruct((n, D), F32),
        )(g2m_efeat, ga, gb, W1a, r2(edge_b1), edge_W2, r2(edge_b2),
          r2(edge_g), r2(edge_beta))

    def scatter_half(blo, bhi, ef):
        return pl.kernel(
            functools.partial(_sc_scatter_body, blo, bhi),
            out_type=jax.ShapeDtypeStruct((NC, ND, D), F32),
            mesh=mesh,
            scratch_types=[pltpu.VMEM((CB,), jnp.int32),
                           pltpu.VMEM((CB, D), F32),
                           pltpu.VMEM((CB,), jnp.int32),
                           pltpu.VMEM((CB, D), F32),
                           pltpu.VMEM((RB, D), F32),
                           pltpu.VMEM_SHARED((ND, D), F32),
                           pltpu.SemaphoreType.DMA,
                           pltpu.SemaphoreType.DMA],
        )(dst_idx, ef)

    ga0, gb0 = gather_half(0, SPLIT)
    ga1, gb1 = gather_half(SPLIT, NBLK)
    ef0 = edge_half(0, SPLIT, ga0, gb0)
    ef1 = edge_half(SPLIT, NBLK, ga1, gb1)
    p0 = scatter_half(0, SPLIT, ef0)
    p1 = scatter_half(SPLIT, NBLK, ef1)

    DBK = 1000
    mesh_out = pl.pallas_call(
        _dst_body,
        grid=(ND // DBK,),
        in_specs=[pl.BlockSpec((NC, DBK, D), lambda i: (0, i, 0)),
                  pl.BlockSpec((NC, DBK, D), lambda i: (0, i, 0)),
                  pl.BlockSpec((DBK, D), lambda i: (i, 0)),
                  _full((D, H)), _full((D, H)), _full((1, H)),
                  _full((H, D)), _full((1, D)), _full((1, D)),
                  _full((1, D))],
        out_specs=pl.BlockSpec((DBK, D), lambda i: (i, 0)),
        out_shape=jax.ShapeDtypeStruct((ND, D), F32),
    )(p0, p1, mesh_nfeat, dW1a, dW1b, r2(dst_b1), dst_W2, r2(dst_b2),
      r2(dst_g), r2(dst_beta))

    return (grid_out, mesh_out)


# speedup vs baseline: 1.7261x; 1.1035x over previous
import functools

import jax
import jax.numpy as jnp
from jax import lax
from jax.experimental import pallas as pl
from jax.experimental.pallas import tpu as pltpu
from jax.experimental.pallas import tpu_sc as plsc

F32 = jnp.float32

E, NS, ND, D, H = 160000, 50000, 10000, 128, 128

NC, NSUB = 2, 16
NW = NC * NSUB

CB = 128
NBLK = E // CB
SPLIT = 640
RB = 80
NRB = ND // RB
RJ = (NRB + NSUB - 1) // NSUB


def _silu(x):
    return x * jax.nn.sigmoid(x)


def _ln(y, g, b):
    m = jnp.mean(y, axis=-1, keepdims=True)
    v = jnp.mean((y - m) ** 2, axis=-1, keepdims=True)
    return (y - m) * lax.rsqrt(v + 1e-5) * g + b


def _grid_body(x_ref, w1b_ref, sw1_ref, sb1_ref, sw2_ref, sb2_ref, sg_ref,
               sbeta_ref, gp_ref, go_ref):
    x = x_ref[...]
    gp_ref[...] = jnp.dot(x, w1b_ref[...], preferred_element_type=F32)
    h = _silu(jnp.dot(x, sw1_ref[...], preferred_element_type=F32)
              + sb1_ref[...])
    y = jnp.dot(h, sw2_ref[...], preferred_element_type=F32) + sb2_ref[...]
    go_ref[...] = x + _ln(y, sg_ref[...], sbeta_ref[...])


def _meshproj_body(x_ref, w1c_ref, mp_ref):
    mp_ref[...] = jnp.dot(x_ref[...], w1c_ref[...],
                          preferred_element_type=F32)


def _edge_body(g2m_ref, ga_ref, w1a_ref, b1_ref, w2_ref, b2_ref,
               g_ref, beta_ref, out_ref):
    x = (jnp.dot(g2m_ref[...], w1a_ref[...], preferred_element_type=F32)
         + ga_ref[...] + b1_ref[...])
    h = _silu(x)
    y = jnp.dot(h, w2_ref[...], preferred_element_type=F32) + b2_ref[...]
    out_ref[...] = _ln(y, g_ref[...], beta_ref[...])


def _dst_body(p0_ref, p1_ref, mesh_ref, dw1a_ref, dw1b_ref, db1_ref,
              dw2_ref, db2_ref, dg_ref, dbeta_ref, out_ref):
    p0 = p0_ref[...]
    p1 = p1_ref[...]
    agg = p0[0] + p0[1] + p1[0] + p1[1]
    mesh = mesh_ref[...]
    h = _silu(jnp.dot(agg, dw1a_ref[...], preferred_element_type=F32)
              + jnp.dot(mesh, dw1b_ref[...], preferred_element_type=F32)
              + db1_ref[...])
    y = jnp.dot(h, dw2_ref[...], preferred_element_type=F32) + db2_ref[...]
    out_ref[...] = mesh + _ln(y, dg_ref[...], dbeta_ref[...])


def _full(shape):
    nd = len(shape)
    return pl.BlockSpec(shape, lambda i: (0,) * nd)


def _sc_gather_body(blo, bhi, sidx_hbm, didx_hbm, gp_hbm, mp_hbm,
                    outg_hbm,
                    si0, di0, ga0, gb0, ii0, si1, di1, ga1, gb1, ii1,
                    sh,
                    smi0, smg0, smo0, smi1, smg1, smo1):
    c = lax.axis_index("c")
    s = lax.axis_index("s")
    wid = s * NC + c
    slots = ((si0, di0, ga0, gb0, ii0, smi0, smg0, smo0),
             (si1, di1, ga1, gb1, ii1, smi1, smg1, smo1))
    nblk = bhi - blo
    gj2 = (nblk + 2 * NW - 1) // (2 * NW)

    for sl in (0, 1):
        iiv = slots[sl][4]
        base = (sl * NSUB + s) * CB
        for k in range(CB // 16):
            iiv[pl.ds(k * 16, 16)] = lax.iota(jnp.int32, 16) + (base + 16 * k)

    def shreg(sl):
        return sh.at[pl.ds((sl * NSUB + s) * CB, CB)]

    def blk_of(q, sl):
        return (q * 2 + sl) * NW + wid

    def fire_idx(q, sl):
        siv, div, _, _, _, smi, _, _ = slots[sl]
        blk = blk_of(q, sl)

        @pl.when(blk < nblk)
        def _():
            base = (blo + blk) * CB
            pltpu.async_copy(sidx_hbm.at[pl.ds(base, CB)], siv, smi)
            pltpu.async_copy(didx_hbm.at[pl.ds(base, CB)], div, smi)

    fire_idx(0, 0)
    fire_idx(0, 1)

    def body(q, carry):
        for sl in (0, 1):
            siv, div, gav, gbv, iiv, smi, smg, smo = slots[sl]
            blk = blk_of(q, sl)
            prev = blk - 2 * NW

            @pl.when(jnp.logical_and(prev >= 0, prev < nblk))
            def _():
                pltpu.make_async_copy(shreg(sl), outg_hbm.at[pl.ds(0, CB)],
                                      smo).wait()

            @pl.when(blk < nblk)
            def _():
                pltpu.make_async_copy(sidx_hbm.at[pl.ds(0, CB)], siv,
                                      smi).wait()
                pltpu.make_async_copy(didx_hbm.at[pl.ds(0, CB)], div,
                                      smi).wait()
                pltpu.async_copy(gp_hbm.at[siv], gav, smg)
                pltpu.async_copy(mp_hbm.at[div], gbv, smg)

        for sl in (0, 1):
            siv, div, gav, gbv, iiv, smi, smg, smo = slots[sl]
            blk = blk_of(q, sl)

            @pl.when(blk < nblk)
            def _():
                base = blk * CB
                pltpu.make_async_copy(gp_hbm.at[siv], gav, smg).wait()
                pltpu.make_async_copy(mp_hbm.at[div], gbv, smg).wait()
                pltpu.sync_copy(gav, shreg(sl))
                pltpu.sync_copy(gbv, sh.at[iiv], add=True)
                pltpu.async_copy(shreg(sl), outg_hbm.at[pl.ds(base, CB)],
                                 smo)

            fire_idx(q + 1, sl)
        return carry

    lax.fori_loop(0, gj2, body, 0)

    for sl in (0, 1):
        siv, div, gav, gbv, iiv, smi, smg, smo = slots[sl]
        blk = blk_of(gj2 - 1, sl)

        @pl.when(blk < nblk)
        def _():
            pltpu.make_async_copy(shreg(sl), outg_hbm.at[pl.ds(0, CB)],
                                  smo).wait()


def _sc_scatter_body(blo, bhi, didx_hbm, ef_hbm, out_hbm,
                     idx0, rows0, idx1, rows1, zbuf_v, acc_sh, smf0, smf1):
    c = lax.axis_index("c")
    s = lax.axis_index("s")
    slots = ((idx0, rows0, smf0), (idx1, rows1, smf1))
    percore = (bhi - blo) // NC
    sj2 = (percore + 2 * NSUB - 1) // (2 * NSUB)

    zero = jnp.zeros((16,), F32)

    def zrow(r, carry):
        for k in range(8):
            zbuf_v[r, pl.ds(k * 16, 16)] = zero
        return carry

    lax.fori_loop(0, RB, zrow, 0)

    def zcopy(q, carry):
        rblk = q * NSUB + s

        @pl.when(rblk < NRB)
        def _():
            pltpu.sync_copy(zbuf_v, acc_sh.at[pl.ds(rblk * RB, RB)])

        return carry

    lax.fori_loop(0, RJ, zcopy, 0)
    plsc.subcore_barrier()

    def blk_of(q, sl):
        return (q * 2 + sl) * NSUB + s

    def fire_fetch(q, sl):
        idxv, rowsv, smf = slots[sl]
        t = blk_of(q, sl)

        @pl.when(t < percore)
        def _():
            local = c * percore + t
            pltpu.async_copy(
                didx_hbm.at[pl.ds((blo + local) * CB, CB)], idxv, smf)
            pltpu.async_copy(ef_hbm.at[pl.ds(local * CB, CB)], rowsv, smf)

    fire_fetch(0, 0)
    fire_fetch(0, 1)

    def body(q, carry):
        for sl in (0, 1):
            idxv, rowsv, smf = slots[sl]
            t = blk_of(q, sl)

            @pl.when(t < percore)
            def _():
                pltpu.make_async_copy(didx_hbm.at[pl.ds(0, CB)], idxv,
                                      smf).wait()
                pltpu.make_async_copy(ef_hbm.at[pl.ds(0, CB)], rowsv,
                                      smf).wait()
                pltpu.sync_copy(rowsv, acc_sh.at[idxv], add=True)

            fire_fetch(q + 1, sl)
        return carry

    lax.fori_loop(0, sj2, body, 0)
    plsc.subcore_barrier()

    def drain(q, carry):
        rblk = q * NSUB + s

        @pl.when(rblk < NRB)
        def _():
            pltpu.sync_copy(acc_sh.at[pl.ds(rblk * RB, RB)],
                            out_hbm.at[c].at[pl.ds(rblk * RB, RB)])

        return carry

    lax.fori_loop(0, RJ, drain, 0)


def kernel(g2m_efeat, grid_nfeat, mesh_nfeat, src_idx, dst_idx,
           edge_W1, edge_b1, edge_W2, edge_b2, edge_g, edge_beta,
           dst_W1, dst_b1, dst_W2, dst_b2, dst_g, dst_beta,
           src_W1, src_b1, src_W2, src_b2, src_g, src_beta):
    W1a, W1b, W1c = edge_W1[:D], edge_W1[D:2 * D], edge_W1[2 * D:]
    dW1a, dW1b = dst_W1[:D], dst_W1[D:]
    r2 = lambda v: v.reshape(1, D)

    GB = 2000
    grid_proj, grid_out = pl.pallas_call(
        _grid_body,
        grid=(NS // GB,),
        in_specs=[pl.BlockSpec((GB, D), lambda i: (i, 0)),
                  _full((D, H)), _full((D, H)), _full((1, H)),
                  _full((H, D)), _full((1, D)), _full((1, D)),
                  _full((1, D))],
        out_specs=[pl.BlockSpec((GB, H), lambda i: (i, 0)),
                   pl.BlockSpec((GB, D), lambda i: (i, 0))],
        out_shape=[jax.ShapeDtypeStruct((NS, H), F32),
                   jax.ShapeDtypeStruct((NS, D), F32)],
    )(grid_nfeat, W1b, src_W1, r2(src_b1), src_W2, r2(src_b2),
      r2(src_g), r2(src_beta))

    MB = 1000
    mesh_proj = pl.pallas_call(
        _meshproj_body,
        grid=(ND // MB,),
        in_specs=[pl.BlockSpec((MB, D), lambda i: (i, 0)), _full((D, H))],
        out_specs=pl.BlockSpec((MB, H), lambda i: (i, 0)),
        out_shape=jax.ShapeDtypeStruct((ND, H), F32),
    )(mesh_nfeat, W1c)

    mesh = plsc.VectorSubcoreMesh(core_axis_name="c", subcore_axis_name="s",
                                  num_cores=NC, num_subcores=NSUB)

    def gather_half(blo, bhi):
        n = (bhi - blo) * CB
        return pl.kernel(
            functools.partial(_sc_gather_body, blo, bhi),
            out_type=jax.ShapeDtypeStruct((n, H), F32),
            mesh=mesh,
            scratch_types=[pltpu.VMEM((CB,), jnp.int32),
                           pltpu.VMEM((CB,), jnp.int32),
                           pltpu.VMEM((CB, H), F32),
                           pltpu.VMEM((CB, H), F32),
                           pltpu.VMEM((CB,), jnp.int32),
                           pltpu.VMEM((CB,), jnp.int32),
                           pltpu.VMEM((CB,), jnp.int32),
                           pltpu.VMEM((CB, H), F32),
                           pltpu.VMEM((CB, H), F32),
                           pltpu.VMEM((CB,), jnp.int32),
                           pltpu.VMEM_SHARED((2 * NSUB * CB, H), F32)]
                          + [pltpu.SemaphoreType.DMA] * 6,
        )(src_idx, dst_idx, grid_proj, mesh_proj)

    EB = 1280

    def edge_half(blo, bhi, ga):
        n = (bhi - blo) * CB
        goff = blo * CB // EB
        return pl.pallas_call(
            _edge_body,
            grid=(n // EB,),
            in_specs=[pl.BlockSpec((EB, D), lambda i: (i + goff, 0)),
                      pl.BlockSpec((EB, H), lambda i: (i, 0)),
                      _full((D, H)), _full((1, H)), _full((H, D)),
                      _full((1, D)), _full((1, D)), _full((1, D))],
            out_specs=pl.BlockSpec((EB, D), lambda i: (i, 0)),
            out_shape=jax.ShapeDtypeStruct((n, D), F32),
        )(g2m_efeat, ga, W1a, r2(edge_b1), edge_W2, r2(edge_b2),
          r2(edge_g), r2(edge_beta))

    def scatter_half(blo, bhi, ef):
        return pl.kernel(
            functools.partial(_sc_scatter_body, blo, bhi),
            out_type=jax.ShapeDtypeStruct((NC, ND, D), F32),
            mesh=mesh,
            scratch_types=[pltpu.VMEM((CB,), jnp.int32),
                           pltpu.VMEM((CB, D), F32),
                           pltpu.VMEM((CB,), jnp.int32),
                           pltpu.VMEM((CB, D), F32),
                           pltpu.VMEM((RB, D), F32),
                           pltpu.VMEM_SHARED((ND, D), F32),
                           pltpu.SemaphoreType.DMA,
                           pltpu.SemaphoreType.DMA],
        )(dst_idx, ef)

    ga0 = gather_half(0, SPLIT)
    ga1 = gather_half(SPLIT, NBLK)
    ef0 = edge_half(0, SPLIT, ga0)
    ef1 = edge_half(SPLIT, NBLK, ga1)
    p0 = scatter_half(0, SPLIT, ef0)
    p1 = scatter_half(SPLIT, NBLK, ef1)

    DBK = 1000
    mesh_out = pl.pallas_call(
        _dst_body,
        grid=(ND // DBK,),
        in_specs=[pl.BlockSpec((NC, DBK, D), lambda i: (0, i, 0)),
                  pl.BlockSpec((NC, DBK, D), lambda i: (0, i, 0)),
                  pl.BlockSpec((DBK, D), lambda i: (i, 0)),
                  _full((D, H)), _full((D, H)), _full((1, H)),
                  _full((H, D)), _full((1, D)), _full((1, D)),
                  _full((1, D))],
        out_specs=pl.BlockSpec((DBK, D), lambda i: (i, 0)),
        out_shape=jax.ShapeDtypeStruct((ND, D), F32),
    )(p0, p1, mesh_nfeat, dW1a, dW1b, r2(dst_b1), dst_W2, r2(dst_b2),
      r2(dst_g), r2(dst_beta))

    return (grid_out, mesh_out)


# speedup vs baseline: 1.7592x; 1.0192x over previous
import functools

import jax
import jax.numpy as jnp
from jax import lax
from jax.experimental import pallas as pl
from jax.experimental.pallas import tpu as pltpu
from jax.experimental.pallas import tpu_sc as plsc

F32 = jnp.float32

E, NS, ND, D, H = 160000, 50000, 10000, 128, 128

NC, NSUB = 2, 16
NW = NC * NSUB

CB = 128
NBLK = E // CB
SPLIT = 640
RB = 80
NRB = ND // RB
RJ = (NRB + NSUB - 1) // NSUB


def _silu(x):
    return x * jax.nn.sigmoid(x)


def _ln(y, g, b):
    m = jnp.mean(y, axis=-1, keepdims=True)
    v = jnp.mean((y - m) ** 2, axis=-1, keepdims=True)
    return (y - m) * lax.rsqrt(v + 1e-5) * g + b


def _srcmlp_body(x_ref, sw1_ref, sb1_ref, sw2_ref, sb2_ref, sg_ref,
                 sbeta_ref, go_ref):
    x = x_ref[...]
    h = _silu(jnp.dot(x, sw1_ref[...], preferred_element_type=F32)
              + sb1_ref[...])
    y = jnp.dot(h, sw2_ref[...], preferred_element_type=F32) + sb2_ref[...]
    go_ref[...] = x + _ln(y, sg_ref[...], sbeta_ref[...])


def _meshproj_body(x_ref, w1c_ref, mp_ref):
    mp_ref[...] = jnp.dot(x_ref[...], w1c_ref[...],
                          preferred_element_type=F32)


def _edge_body(g2m_ref, ga_ref, w1a_ref, b1_ref, w2_ref, b2_ref,
               g_ref, beta_ref, out_ref):
    x = (jnp.dot(g2m_ref[...], w1a_ref[...], preferred_element_type=F32)
         + ga_ref[...] + b1_ref[...])
    h = _silu(x)
    y = jnp.dot(h, w2_ref[...], preferred_element_type=F32) + b2_ref[...]
    out_ref[...] = _ln(y, g_ref[...], beta_ref[...])


def _dst_body(p0_ref, p1_ref, mesh_ref, dw1a_ref, dw1b_ref, db1_ref,
              dw2_ref, db2_ref, dg_ref, dbeta_ref, out_ref):
    p0 = p0_ref[...]
    p1 = p1_ref[...]
    agg = p0[0] + p0[1] + p1[0] + p1[1]
    mesh = mesh_ref[...]
    h = _silu(jnp.dot(agg, dw1a_ref[...], preferred_element_type=F32)
              + jnp.dot(mesh, dw1b_ref[...], preferred_element_type=F32)
              + db1_ref[...])
    y = jnp.dot(h, dw2_ref[...], preferred_element_type=F32) + db2_ref[...]
    out_ref[...] = mesh + _ln(y, dg_ref[...], dbeta_ref[...])


def _full(shape):
    nd = len(shape)
    return pl.BlockSpec(shape, lambda i: (0,) * nd)


def _sc_gather_body(blo, bhi, sidx_hbm, didx_hbm, gp_hbm, mp_hbm,
                    outg_hbm,
                    si0, di0, ga0, gb0, ii0, si1, di1, ga1, gb1, ii1,
                    sh,
                    smi0, smg0, smo0, smi1, smg1, smo1):
    c = lax.axis_index("c")
    s = lax.axis_index("s")
    wid = s * NC + c
    slots = ((si0, di0, ga0, gb0, ii0, smi0, smg0, smo0),
             (si1, di1, ga1, gb1, ii1, smi1, smg1, smo1))
    nblk = bhi - blo
    gj2 = (nblk + 2 * NW - 1) // (2 * NW)

    for sl in (0, 1):
        iiv = slots[sl][4]
        base = (sl * NSUB + s) * CB
        for k in range(CB // 16):
            iiv[pl.ds(k * 16, 16)] = lax.iota(jnp.int32, 16) + (base + 16 * k)

    def shreg(sl):
        return sh.at[pl.ds((sl * NSUB + s) * CB, CB)]

    def blk_of(q, sl):
        return (q * 2 + sl) * NW + wid

    def fire_idx(q, sl):
        siv, div, _, _, _, smi, _, _ = slots[sl]
        blk = blk_of(q, sl)

        @pl.when(blk < nblk)
        def _():
            base = (blo + blk) * CB
            pltpu.async_copy(sidx_hbm.at[pl.ds(base, CB)], siv, smi)
            pltpu.async_copy(didx_hbm.at[pl.ds(base, CB)], div, smi)

    fire_idx(0, 0)
    fire_idx(0, 1)

    def body(q, carry):
        for sl in (0, 1):
            siv, div, gav, gbv, iiv, smi, smg, smo = slots[sl]
            blk = blk_of(q, sl)
            prev = blk - 2 * NW

            @pl.when(jnp.logical_and(prev >= 0, prev < nblk))
            def _():
                pltpu.make_async_copy(shreg(sl), outg_hbm.at[pl.ds(0, CB)],
                                      smo).wait()

            @pl.when(blk < nblk)
            def _():
                pltpu.make_async_copy(sidx_hbm.at[pl.ds(0, CB)], siv,
                                      smi).wait()
                pltpu.make_async_copy(didx_hbm.at[pl.ds(0, CB)], div,
                                      smi).wait()
                pltpu.async_copy(gp_hbm.at[siv], gav, smg)
                pltpu.async_copy(mp_hbm.at[div], gbv, smg)

        for sl in (0, 1):
            siv, div, gav, gbv, iiv, smi, smg, smo = slots[sl]
            blk = blk_of(q, sl)

            @pl.when(blk < nblk)
            def _():
                base = blk * CB
                pltpu.make_async_copy(gp_hbm.at[siv], gav, smg).wait()
                pltpu.make_async_copy(mp_hbm.at[div], gbv, smg).wait()
                pltpu.sync_copy(gav, shreg(sl))
                pltpu.sync_copy(gbv, sh.at[iiv], add=True)
                pltpu.async_copy(shreg(sl), outg_hbm.at[pl.ds(base, CB)],
                                 smo)

            fire_idx(q + 1, sl)
        return carry

    lax.fori_loop(0, gj2, body, 0)

    for sl in (0, 1):
        siv, div, gav, gbv, iiv, smi, smg, smo = slots[sl]
        blk = blk_of(gj2 - 1, sl)

        @pl.when(blk < nblk)
        def _():
            pltpu.make_async_copy(shreg(sl), outg_hbm.at[pl.ds(0, CB)],
                                  smo).wait()


def _sc_scatter_body(blo, bhi, didx_hbm, ef_hbm, out_hbm,
                     idx0, rows0, idx1, rows1, zbuf_v, acc_sh, smf0, smf1):
    c = lax.axis_index("c")
    s = lax.axis_index("s")
    slots = ((idx0, rows0, smf0), (idx1, rows1, smf1))
    percore = (bhi - blo) // NC
    sj2 = (percore + 2 * NSUB - 1) // (2 * NSUB)

    zero = jnp.zeros((16,), F32)

    def zrow(r, carry):
        for k in range(8):
            zbuf_v[r, pl.ds(k * 16, 16)] = zero
        return carry

    lax.fori_loop(0, RB, zrow, 0)

    def zcopy(q, carry):
        rblk = q * NSUB + s

        @pl.when(rblk < NRB)
        def _():
            pltpu.sync_copy(zbuf_v, acc_sh.at[pl.ds(rblk * RB, RB)])

        return carry

    lax.fori_loop(0, RJ, zcopy, 0)
    plsc.subcore_barrier()

    def blk_of(q, sl):
        return (q * 2 + sl) * NSUB + s

    def fire_fetch(q, sl):
        idxv, rowsv, smf = slots[sl]
        t = blk_of(q, sl)

        @pl.when(t < percore)
        def _():
            local = c * percore + t
            pltpu.async_copy(
                didx_hbm.at[pl.ds((blo + local) * CB, CB)], idxv, smf)
            pltpu.async_copy(ef_hbm.at[pl.ds(local * CB, CB)], rowsv, smf)

    fire_fetch(0, 0)
    fire_fetch(0, 1)

    def body(q, carry):
        for sl in (0, 1):
            idxv, rowsv, smf = slots[sl]
            t = blk_of(q, sl)

            @pl.when(t < percore)
            def _():
                pltpu.make_async_copy(didx_hbm.at[pl.ds(0, CB)], idxv,
                                      smf).wait()
                pltpu.make_async_copy(ef_hbm.at[pl.ds(0, CB)], rowsv,
                                      smf).wait()
                pltpu.sync_copy(rowsv, acc_sh.at[idxv], add=True)

            fire_fetch(q + 1, sl)
        return carry

    lax.fori_loop(0, sj2, body, 0)
    plsc.subcore_barrier()

    def drain(q, carry):
        rblk = q * NSUB + s

        @pl.when(rblk < NRB)
        def _():
            pltpu.sync_copy(acc_sh.at[pl.ds(rblk * RB, RB)],
                            out_hbm.at[c].at[pl.ds(rblk * RB, RB)])

        return carry

    lax.fori_loop(0, RJ, drain, 0)


def kernel(g2m_efeat, grid_nfeat, mesh_nfeat, src_idx, dst_idx,
           edge_W1, edge_b1, edge_W2, edge_b2, edge_g, edge_beta,
           dst_W1, dst_b1, dst_W2, dst_b2, dst_g, dst_beta,
           src_W1, src_b1, src_W2, src_b2, src_g, src_beta):
    W1a, W1b, W1c = edge_W1[:D], edge_W1[D:2 * D], edge_W1[2 * D:]
    dW1a, dW1b = dst_W1[:D], dst_W1[D:]
    r2 = lambda v: v.reshape(1, D)

    GB = 2000
    grid_proj = pl.pallas_call(
        _meshproj_body,
        grid=(NS // GB,),
        in_specs=[pl.BlockSpec((GB, D), lambda i: (i, 0)), _full((D, H))],
        out_specs=pl.BlockSpec((GB, H), lambda i: (i, 0)),
        out_shape=jax.ShapeDtypeStruct((NS, H), F32),
    )(grid_nfeat, W1b)

    MB = 1000
    mesh_proj = pl.pallas_call(
        _meshproj_body,
        grid=(ND // MB,),
        in_specs=[pl.BlockSpec((MB, D), lambda i: (i, 0)), _full((D, H))],
        out_specs=pl.BlockSpec((MB, H), lambda i: (i, 0)),
        out_shape=jax.ShapeDtypeStruct((ND, H), F32),
    )(mesh_nfeat, W1c)

    mesh = plsc.VectorSubcoreMesh(core_axis_name="c", subcore_axis_name="s",
                                  num_cores=NC, num_subcores=NSUB)

    def gather_half(blo, bhi):
        n = (bhi - blo) * CB
        return pl.kernel(
            functools.partial(_sc_gather_body, blo, bhi),
            out_type=jax.ShapeDtypeStruct((n, H), F32),
            mesh=mesh,
            scratch_types=[pltpu.VMEM((CB,), jnp.int32),
                           pltpu.VMEM((CB,), jnp.int32),
                           pltpu.VMEM((CB, H), F32),
                           pltpu.VMEM((CB, H), F32),
                           pltpu.VMEM((CB,), jnp.int32),
                           pltpu.VMEM((CB,), jnp.int32),
                           pltpu.VMEM((CB,), jnp.int32),
                           pltpu.VMEM((CB, H), F32),
                           pltpu.VMEM((CB, H), F32),
                           pltpu.VMEM((CB,), jnp.int32),
                           pltpu.VMEM_SHARED((2 * NSUB * CB, H), F32)]
                          + [pltpu.SemaphoreType.DMA] * 6,
        )(src_idx, dst_idx, grid_proj, mesh_proj)

    EB = 1280

    def edge_half(blo, bhi, ga):
        n = (bhi - blo) * CB
        goff = blo * CB // EB
        return pl.pallas_call(
            _edge_body,
            grid=(n // EB,),
            in_specs=[pl.BlockSpec((EB, D), lambda i: (i + goff, 0)),
                      pl.BlockSpec((EB, H), lambda i: (i, 0)),
                      _full((D, H)), _full((1, H)), _full((H, D)),
                      _full((1, D)), _full((1, D)), _full((1, D))],
            out_specs=pl.BlockSpec((EB, D), lambda i: (i, 0)),
            out_shape=jax.ShapeDtypeStruct((n, D), F32),
        )(g2m_efeat, ga, W1a, r2(edge_b1), edge_W2, r2(edge_b2),
          r2(edge_g), r2(edge_beta))

    def scatter_half(blo, bhi, ef):
        return pl.kernel(
            functools.partial(_sc_scatter_body, blo, bhi),
            out_type=jax.ShapeDtypeStruct((NC, ND, D), F32),
            mesh=mesh,
            scratch_types=[pltpu.VMEM((CB,), jnp.int32),
                           pltpu.VMEM((CB, D), F32),
                           pltpu.VMEM((CB,), jnp.int32),
                           pltpu.VMEM((CB, D), F32),
                           pltpu.VMEM((RB, D), F32),
                           pltpu.VMEM_SHARED((ND, D), F32),
                           pltpu.SemaphoreType.DMA,
                           pltpu.SemaphoreType.DMA],
        )(dst_idx, ef)

    ga0 = gather_half(0, SPLIT)
    ga1 = gather_half(SPLIT, NBLK)
    grid_out = pl.pallas_call(
        _srcmlp_body,
        grid=(NS // GB,),
        in_specs=[pl.BlockSpec((GB, D), lambda i: (i, 0)),
                  _full((D, H)), _full((1, H)), _full((H, D)),
                  _full((1, D)), _full((1, D)), _full((1, D))],
        out_specs=pl.BlockSpec((GB, D), lambda i: (i, 0)),
        out_shape=jax.ShapeDtypeStruct((NS, D), F32),
    )(grid_nfeat, src_W1, r2(src_b1), src_W2, r2(src_b2),
      r2(src_g), r2(src_beta))
    ef0 = edge_half(0, SPLIT, ga0)
    ef1 = edge_half(SPLIT, NBLK, ga1)
    p0 = scatter_half(0, SPLIT, ef0)
    p1 = scatter_half(SPLIT, NBLK, ef1)

    DBK = 1000
    mesh_out = pl.pallas_call(
        _dst_body,
        grid=(ND // DBK,),
        in_specs=[pl.BlockSpec((NC, DBK, D), lambda i: (0, i, 0)),
                  pl.BlockSpec((NC, DBK, D), lambda i: (0, i, 0)),
                  pl.BlockSpec((DBK, D), lambda i: (i, 0)),
                  _full((D, H)), _full((D, H)), _full((1, H)),
                  _full((H, D)), _full((1, D)), _full((1, D)),
                  _full((1, D))],
        out_specs=pl.BlockSpec((DBK, D), lambda i: (i, 0)),
        out_shape=jax.ShapeDtypeStruct((ND, D), F32),
    )(p0, p1, mesh_nfeat, dW1a, dW1b, r2(dst_b1), dst_W2, r2(dst_b2),
      r2(dst_g), r2(dst_beta))

    return (grid_out, mesh_out)
